# baseline (speedup 1.0000x reference)
import jax
import jax.numpy as jnp
from jax.experimental import pallas as pl

_DIM = 384
_EPS = 1e-10
_B = 32
_HW = 1024


def _vq_kernel(x_ref, quant_ref, embed_ref, hist_ref, qsum_ref, perp_ref, div_ref):
    i = pl.program_id(0)

    xb = x_ref[0]
    xr = jnp.maximum(xb, 0.0)
    s = jnp.sum(xr, axis=0, keepdims=True)
    r = 1.0 / (s + _EPS)
    xn = xr * r

    m = jnp.max(xr, axis=0, keepdims=True)
    iota = jax.lax.broadcasted_iota(jnp.int32, (_DIM, _HW), 0)
    inds = jnp.min(jnp.where(xr == m, iota, _DIM), axis=0, keepdims=True)

    one_hot = (iota == inds).astype(jnp.float32)
    quant_ref[0] = one_hot
    embed_ref[0] = inds

    hist_part = jnp.sum(one_hot, axis=1, keepdims=True)
    qsum_part = jnp.sum(xn, axis=1, keepdims=True)

    @pl.when(i == 0)
    def _init():
        hist_ref[...] = hist_part
        qsum_ref[...] = qsum_part

    @pl.when(i > 0)
    def _acc():
        hist_ref[...] += hist_part
        qsum_ref[...] += qsum_part

    @pl.when(i == _B - 1)
    def _finalize():
        n = float(_B * _HW)
        avg_probs = hist_ref[...] / n
        ent = jnp.sum(avg_probs * jnp.log(avg_probs + 1e-10), axis=0, keepdims=True)
        perp_ref[...] = jnp.exp(-ent)
        q_bar = qsum_ref[...] / n
        div_ref[...] = jnp.mean((q_bar * float(_DIM) - 1.0) ** 2, axis=0, keepdims=True)


def kernel(x):
    b, dim, h, w = x.shape
    xr = x.reshape(b, dim, h * w)

    quant, embed, _hist, _qsum, perp, div = pl.pallas_call(
        _vq_kernel,
        grid=(b,),
        in_specs=[pl.BlockSpec((1, dim, h * w), lambda i: (i, 0, 0))],
        out_specs=[
            pl.BlockSpec((1, dim, h * w), lambda i: (i, 0, 0)),
            pl.BlockSpec((1, 1, h * w), lambda i: (i, 0, 0)),
            pl.BlockSpec((dim, 1), lambda i: (0, 0)),
            pl.BlockSpec((dim, 1), lambda i: (0, 0)),
            pl.BlockSpec((1, 1), lambda i: (0, 0)),
            pl.BlockSpec((1, 1), lambda i: (0, 0)),
        ],
        out_shape=[
            jax.ShapeDtypeStruct((b, dim, h * w), jnp.float32),
            jax.ShapeDtypeStruct((b, 1, h * w), jnp.int32),
            jax.ShapeDtypeStruct((dim, 1), jnp.float32),
            jax.ShapeDtypeStruct((dim, 1), jnp.float32),
            jax.ShapeDtypeStruct((1, 1), jnp.float32),
            jax.ShapeDtypeStruct((1, 1), jnp.float32),
        ],
    )(xr)

    quantize = quant.reshape(b, dim, h, w)
    embed_ind = embed.reshape(b, h, w)
    return (quantize, div[0, 0], embed_ind, perp[0, 0])
